# baseline (speedup 1.0000x reference)
import jax
import jax.numpy as jnp
from jax import lax
from jax.experimental import pallas as pl
from jax.experimental.pallas import tpu as pltpu
from jax.experimental.pallas import tpu_sc as plsc

N = 10000
E = 320000
D = 128
SEM_H = 128
R = 2

NC = 2
NS = 16
NW = NC * NS
L = 16

EPW = E // NW
KB = 80
NB = EPW // KB
VB = KB // L

STRIPE = N // NS
DSTRIPE = 640
ND = NS * DSTRIPE

CH = 80
NCH = N // CH
CPW = 4
NP = N

_f32 = jnp.float32


def _proj_body(dstf, srcA, srcB, WA, alA, arA, WB, alB, arB,
               hsA_o, hsB_o, sc_o):
    hsA = jnp.dot(srcA[...], WA[...], preferred_element_type=_f32)
    hsB = jnp.dot(srcB[...], WB[...], preferred_element_type=_f32)
    hdA = jnp.dot(dstf[...], WA[...], preferred_element_type=_f32)
    hdB = jnp.dot(dstf[...], WB[...], preferred_element_type=_f32)
    hsA_o[...] = hsA
    hsB_o[...] = hsB
    elA = jnp.sum(hsA * alA[...], axis=1, keepdims=True)
    erA = jnp.sum(hdA * arA[...], axis=1, keepdims=True)
    elB = jnp.sum(hsB * alB[...], axis=1, keepdims=True)
    erB = jnp.sum(hdB * arB[...], axis=1, keepdims=True)
    sc_o[...] = jnp.concatenate([elA, erA, elB, erB], axis=1)


_proj = pl.pallas_call(
    _proj_body,
    out_shape=(
        jax.ShapeDtypeStruct((N, D), _f32),
        jax.ShapeDtypeStruct((N, D), _f32),
        jax.ShapeDtypeStruct((N, 4), _f32),
    ),
)


def _edge_body(hsA, elA, erA, sdA, hsB, elB, erB, sdB,
               accA_o, denA_o, accB_o, denB_o,
               acc_sh, den_sh, el_ts, er_ts, sdw, rows, wbuf, den_idx, zbuf,
               gsem, isem, asem, ssem):
    cid = lax.axis_index("c")
    sid = lax.axis_index("s")
    wid = cid * NS + sid
    zv = jnp.zeros((L,), _f32)

    for hs_hbm, el_hbm, er_hbm, sd_hbm, acc_o, den_o in (
        (hsA, elA, erA, sdA, accA_o, denA_o),
        (hsB, elB, erB, sdB, accB_o, denB_o),
    ):
        pltpu.sync_copy(el_hbm, el_ts)
        pltpu.sync_copy(er_hbm, er_ts)

        @plsc.parallel_loop(0, 2 * KB, step=1, unroll=4)
        def _zr(i):
            for cc in range(D // L):
                rows[i, pl.ds(cc * L, L)] = zv

        def _zz(i, c):
            zbuf[pl.ds(i * L, L)] = zv
            return c
        lax.fori_loop(0, DSTRIPE // L, _zz, 0)

        base = sid * STRIPE
        for t, ln in ((0, 160), (160, 160), (320, 160), (480, 145)):
            pltpu.sync_copy(rows.at[pl.ds(0, ln)],
                            acc_sh.at[pl.ds(base + t, ln)])
        pltpu.sync_copy(zbuf, den_sh.at[pl.ds(sid * DSTRIPE, DSTRIPE)])

        plsc.subcore_barrier()

        pltpu.sync_copy(sd_hbm.at[wid, 0], sdw.at[0])
        pltpu.async_copy(hs_hbm.at[sdw.at[0, 0]], rows.at[pl.ds(0, KB)], gsem)
        pltpu.async_copy(sd_hbm.at[wid, 1], sdw.at[1], isem)

        def _batch(g, c):
            par = lax.rem(g, 2)
            pob = par * KB
            parn = lax.rem(g + 1, 2)

            @pl.when(g + 1 < NB)
            def _():
                pltpu.make_async_copy(sd_hbm.at[wid, g + 1], sdw.at[parn],
                                      isem).wait()

                @pl.when(g >= 1)
                def _():
                    pltpu.make_async_copy(rows.at[pl.ds(parn * KB, KB)],
                                          acc_sh.at[den_idx.at[parn]],
                                          ssem).wait()

                pltpu.async_copy(hs_hbm.at[sdw.at[parn, 0]],
                                 rows.at[pl.ds(parn * KB, KB)], gsem)

            pltpu.make_async_copy(hs_hbm.at[sdw.at[par, 0]],
                                  rows.at[pl.ds(pob, KB)], gsem).wait()

            @pl.when(g >= 2)
            def _():
                pltpu.make_async_copy(wbuf.at[par],
                                      den_sh.at[den_idx.at[par]], asem).wait()

            for v in range(VB):
                sidx = sdw[par, 0, pl.ds(v * L, L)]
                didx = sdw[par, 1, pl.ds(v * L, L)]
                e = plsc.load_gather(el_ts, [sidx]) + plsc.load_gather(er_ts, [didx])
                e = jnp.where(e > 0, e, _f32(0.2) * e)
                w = jnp.exp(e)
                wbuf[par, pl.ds(v * L, L)] = w
                den_idx[par, pl.ds(v * L, L)] = didx

            pltpu.async_copy(wbuf.at[par], den_sh.at[den_idx.at[par]], asem,
                             add=True)

            @plsc.parallel_loop(0, KB, step=1, unroll=8)
            def _scale(j):
                pv = jnp.broadcast_to(par, (L,)).astype(jnp.int32)
                jv = jnp.broadcast_to(j, (L,)).astype(jnp.int32)
                wsp = plsc.load_gather(wbuf, [pv, jv])
                for cc in range(D // L):
                    sl = pl.ds(cc * L, L)
                    rows[pob + j, sl] = rows[pob + j, sl] * wsp

            pltpu.async_copy(rows.at[pl.ds(pob, KB)],
                             acc_sh.at[den_idx.at[par]], ssem, add=True)

            @pl.when(g + 2 < NB)
            def _():
                pltpu.async_copy(sd_hbm.at[wid, g + 2], sdw.at[par], isem)

            return c
        lax.fori_loop(0, NB, _batch, 0)

        pltpu.make_async_copy(wbuf.at[0], den_sh.at[den_idx.at[0]], asem).wait()
        pltpu.make_async_copy(wbuf.at[1], den_sh.at[den_idx.at[1]], asem).wait()
        pltpu.make_async_copy(rows.at[pl.ds(0, KB)],
                              acc_sh.at[den_idx.at[0]], ssem).wait()
        pltpu.make_async_copy(rows.at[pl.ds(KB, KB)],
                              acc_sh.at[den_idx.at[1]], ssem).wait()
        plsc.subcore_barrier()
        pltpu.sync_copy(acc_sh.at[pl.ds(base, STRIPE)],
                        acc_o.at[cid, pl.ds(base, STRIPE)])

        @pl.when(sid == 0)
        def _():
            pltpu.sync_copy(den_sh, den_o.at[cid])

        plsc.subcore_barrier()


_edge = pl.kernel(
    _edge_body,
    out_type=(
        jax.ShapeDtypeStruct((NC, N, D), _f32),
        jax.ShapeDtypeStruct((NC, ND), _f32),
        jax.ShapeDtypeStruct((NC, N, D), _f32),
        jax.ShapeDtypeStruct((NC, ND), _f32),
    ),
    mesh=plsc.VectorSubcoreMesh(core_axis_name="c", subcore_axis_name="s"),
    compiler_params=pltpu.CompilerParams(use_tc_tiling_on_sc=False, needs_layout_passes=False),
    scratch_types=[
        pltpu.VMEM_SHARED((N, D), _f32),
        pltpu.VMEM_SHARED((ND,), _f32),
        pltpu.VMEM((N,), _f32),
        pltpu.VMEM((N,), _f32),
        pltpu.VMEM((2, 2, KB), jnp.int32),
        pltpu.VMEM((2 * KB, D), _f32),
        pltpu.VMEM((2, KB), _f32),
        pltpu.VMEM((2, KB), jnp.int32),
        pltpu.VMEM((DSTRIPE,), _f32),
        pltpu.SemaphoreType.DMA,
        pltpu.SemaphoreType.DMA,
        pltpu.SemaphoreType.DMA,
        pltpu.SemaphoreType.DMA,
    ],
)


def _fuse_body(accA, dnA, accB, dnB, biasA, biasB, W1, b1, W2, z_o, att_o):
    dA = (dnA[0, :N] + dnA[1, :N]).reshape(N, 1)
    dB = (dnB[0, :N] + dnB[1, :N]).reshape(N, 1)
    dA = jnp.where(dA == 0, _f32(1.0), dA)
    dB = jnp.where(dB == 0, _f32(1.0), dB)
    zA = (accA[0] + accA[1]) / dA + biasA[...]
    zB = (accB[0] + accB[1]) / dB + biasB[...]
    zA = jnp.where(zA > 0, zA, jnp.exp(zA) - _f32(1.0))
    zB = jnp.where(zB > 0, zB, jnp.exp(zB) - _f32(1.0))
    sA = jnp.dot(
        jnp.tanh(jnp.dot(zA, W1[...], preferred_element_type=_f32) + b1[...]),
        W2[...], preferred_element_type=_f32)
    sB = jnp.dot(
        jnp.tanh(jnp.dot(zB, W1[...], preferred_element_type=_f32) + b1[...]),
        W2[...], preferred_element_type=_f32)
    wA = jnp.mean(sA)
    wB = jnp.mean(sB)
    m = jnp.maximum(wA, wB)
    eA = jnp.exp(wA - m)
    eB = jnp.exp(wB - m)
    aA = eA / (eA + eB)
    aB = eB / (eA + eB)
    z_o[...] = aA * zA + aB * zB
    att_o[...] = jnp.concatenate(
        [jnp.broadcast_to(aA, (1, 1)), jnp.broadcast_to(aB, (1, 1))], axis=1)


_fuse = pl.pallas_call(
    _fuse_body,
    out_shape=(
        jax.ShapeDtypeStruct((N, D), _f32),
        jax.ShapeDtypeStruct((1, R), _f32),
    ),
)


def kernel(dst_feat, src_feat_A, src_feat_B, edge_index_A, edge_index_B,
           W_gat_A, attn_l_A, attn_r_A, bias_A,
           W_gat_B, attn_l_B, attn_r_B, bias_B,
           W1, b1, W2):
    srcA = edge_index_A[0].astype(jnp.int32).reshape(NW, NB, 1, KB)
    dstA = edge_index_A[1].astype(jnp.int32).reshape(NW, NB, 1, KB)
    srcB = edge_index_B[0].astype(jnp.int32).reshape(NW, NB, 1, KB)
    dstB = edge_index_B[1].astype(jnp.int32).reshape(NW, NB, 1, KB)
    sdA = jnp.concatenate([srcA, dstA], axis=2)
    sdB = jnp.concatenate([srcB, dstB], axis=2)

    hsA, hsB, sc4 = _proj(
        dst_feat, src_feat_A, src_feat_B,
        W_gat_A, attn_l_A.reshape(1, D), attn_r_A.reshape(1, D),
        W_gat_B, attn_l_B.reshape(1, D), attn_r_B.reshape(1, D))
    elA = sc4[:, 0]
    erA = sc4[:, 1]
    elB = sc4[:, 2]
    erB = sc4[:, 3]

    accA, denA, accB, denB = _edge(hsA, elA, erA, sdA,
                                   hsB, elB, erB, sdB)
    z, att = _fuse(accA, denA, accB, denB,
                   bias_A.reshape(1, D), bias_B.reshape(1, D),
                   W1, b1.reshape(1, SEM_H), W2)
    return z, att.reshape(R)


# speedup vs baseline: 1.0134x; 1.0134x over previous
import jax
import jax.numpy as jnp
from jax import lax
from jax.experimental import pallas as pl
from jax.experimental.pallas import tpu as pltpu
from jax.experimental.pallas import tpu_sc as plsc

N = 10000
E = 320000
D = 128
SEM_H = 128
R = 2

NC = 2
NS = 16
NW = NC * NS
L = 16

EPW = E // NW
KB = 80
NB = EPW // KB
VB = KB // L
W = 25
NBW = NB // W

STRIPE = N // NS
DSTRIPE = 640
ND = NS * DSTRIPE

CH = 80
NCH = N // CH
CPW = 4
NP = N

_f32 = jnp.float32


def _proj_body(dstf, srcA, srcB, WA, alA, arA, WB, alB, arB,
               hsA_o, hsB_o, sc_o):
    hsA = jnp.dot(srcA[...], WA[...], preferred_element_type=_f32)
    hsB = jnp.dot(srcB[...], WB[...], preferred_element_type=_f32)
    hdA = jnp.dot(dstf[...], WA[...], preferred_element_type=_f32)
    hdB = jnp.dot(dstf[...], WB[...], preferred_element_type=_f32)
    hsA_o[...] = hsA
    hsB_o[...] = hsB
    elA = jnp.sum(hsA * alA[...], axis=1, keepdims=True)
    erA = jnp.sum(hdA * arA[...], axis=1, keepdims=True)
    elB = jnp.sum(hsB * alB[...], axis=1, keepdims=True)
    erB = jnp.sum(hdB * arB[...], axis=1, keepdims=True)
    sc_o[...] = jnp.concatenate([elA, erA, elB, erB], axis=1)


_proj = pl.pallas_call(
    _proj_body,
    out_shape=(
        jax.ShapeDtypeStruct((N, D), _f32),
        jax.ShapeDtypeStruct((N, D), _f32),
        jax.ShapeDtypeStruct((N, 4), _f32),
    ),
)


def _edge_body(hsA, elA, erA, sdA, hsB, elB, erB, sdB,
               accA_o, denA_o, accB_o, denB_o,
               acc_sh, den_sh, el_ts, er_ts, sdw2, rows, wbuf, zbuf,
               gsem, isem, asem, ssem):
    cid = lax.axis_index("c")
    sid = lax.axis_index("s")
    wid = cid * NS + sid
    zv = jnp.zeros((L,), _f32)

    for hs_hbm, el_hbm, er_hbm, sd_hbm, acc_o, den_o in (
        (hsA, elA, erA, sdA, accA_o, denA_o),
        (hsB, elB, erB, sdB, accB_o, denB_o),
    ):
        pltpu.sync_copy(el_hbm, el_ts)
        pltpu.sync_copy(er_hbm, er_ts)

        @plsc.parallel_loop(0, 2 * KB, step=1, unroll=4)
        def _zr(i):
            for cc in range(D // L):
                rows[i, pl.ds(cc * L, L)] = zv

        def _zz(i, c):
            zbuf[pl.ds(i * L, L)] = zv
            return c
        lax.fori_loop(0, DSTRIPE // L, _zz, 0)

        base = sid * STRIPE
        for t, ln in ((0, 160), (160, 160), (320, 160), (480, 145)):
            pltpu.sync_copy(rows.at[pl.ds(0, ln)],
                            acc_sh.at[pl.ds(base + t, ln)])
        pltpu.sync_copy(zbuf, den_sh.at[pl.ds(sid * DSTRIPE, DSTRIPE)])

        pltpu.sync_copy(sd_hbm.at[wid, pl.ds(0, W)], sdw2.at[0])
        pltpu.async_copy(sd_hbm.at[wid, pl.ds(W, W)], sdw2.at[1], isem)

        plsc.subcore_barrier()

        pltpu.async_copy(hs_hbm.at[sdw2.at[0, 0, 0]], rows.at[pl.ds(0, KB)],
                         gsem)

        def _batch(g, c):
            par = lax.rem(g, 2)
            pob = par * KB
            parn = lax.rem(g + 1, 2)
            k = lax.div(g, W)
            j = lax.rem(g, W)
            kpar = lax.rem(k, 2)
            kparn = 1 - kpar

            @pl.when(g + 1 < NB)
            def _():
                @pl.when(g >= 1)
                def _():
                    bp = jnp.where(j == 0, kparn, kpar)
                    jp = lax.rem(j - 1 + W, W)
                    pltpu.make_async_copy(rows.at[pl.ds(parn * KB, KB)],
                                          acc_sh.at[sdw2.at[bp, jp, 1]],
                                          ssem).wait()

                @pl.when(j == W - 1)
                def _():
                    pltpu.make_async_copy(
                        sd_hbm.at[wid, pl.ds((k + 1) * W, W)],
                        sdw2.at[kparn], isem).wait()

                bn = jnp.where(j == W - 1, kparn, kpar)
                jn = lax.rem(j + 1, W)
                pltpu.async_copy(hs_hbm.at[sdw2.at[bn, jn, 0]],
                                 rows.at[pl.ds(parn * KB, KB)], gsem)

            pltpu.make_async_copy(hs_hbm.at[sdw2.at[kpar, j, 0]],
                                  rows.at[pl.ds(pob, KB)], gsem).wait()

            @pl.when(g >= 2)
            def _():
                bp2 = jnp.where(j <= 1, kparn, kpar)
                jp2 = lax.rem(j - 2 + W, W)
                pltpu.make_async_copy(wbuf.at[par],
                                      den_sh.at[sdw2.at[bp2, jp2, 1]],
                                      asem).wait()

            @pl.when((j == 2) & (k + 1 < NBW) & (k >= 1))
            def _():
                pltpu.async_copy(sd_hbm.at[wid, pl.ds((k + 1) * W, W)],
                                 sdw2.at[kparn], isem)

            for v in range(VB):
                sidx = sdw2[kpar, j, 0, pl.ds(v * L, L)]
                didx = sdw2[kpar, j, 1, pl.ds(v * L, L)]
                e = plsc.load_gather(el_ts, [sidx]) + plsc.load_gather(er_ts, [didx])
                e = jnp.where(e > 0, e, _f32(0.2) * e)
                w = jnp.exp(e)
                wbuf[par, pl.ds(v * L, L)] = w

            pltpu.async_copy(wbuf.at[par], den_sh.at[sdw2.at[kpar, j, 1]],
                             asem, add=True)

            @plsc.parallel_loop(0, KB, step=1, unroll=4)
            def _scale(jj):
                pv = jnp.broadcast_to(par, (L,)).astype(jnp.int32)
                jv = jnp.broadcast_to(jj, (L,)).astype(jnp.int32)
                wsp = plsc.load_gather(wbuf, [pv, jv])
                for cc in range(D // L):
                    sl = pl.ds(cc * L, L)
                    rows[pob + jj, sl] = rows[pob + jj, sl] * wsp

            pltpu.async_copy(rows.at[pl.ds(pob, KB)],
                             acc_sh.at[sdw2.at[kpar, j, 1]], ssem, add=True)

            return c
        lax.fori_loop(0, NB, _batch, 0)

        pltpu.make_async_copy(wbuf.at[0], den_sh.at[sdw2.at[0, W - 1, 1]],
                              asem).wait()
        pltpu.make_async_copy(wbuf.at[1], den_sh.at[sdw2.at[0, W - 2, 1]],
                              asem).wait()
        pltpu.make_async_copy(rows.at[pl.ds(0, KB)],
                              acc_sh.at[sdw2.at[0, W - 1, 1]], ssem).wait()
        pltpu.make_async_copy(rows.at[pl.ds(KB, KB)],
                              acc_sh.at[sdw2.at[0, W - 2, 1]], ssem).wait()
        plsc.subcore_barrier()
        pltpu.sync_copy(acc_sh.at[pl.ds(base, STRIPE)],
                        acc_o.at[cid, pl.ds(base, STRIPE)])

        @pl.when(sid == 0)
        def _():
            pltpu.sync_copy(den_sh, den_o.at[cid])

        plsc.subcore_barrier()


_edge = pl.kernel(
    _edge_body,
    out_type=(
        jax.ShapeDtypeStruct((NC, N, D), _f32),
        jax.ShapeDtypeStruct((NC, ND), _f32),
        jax.ShapeDtypeStruct((NC, N, D), _f32),
        jax.ShapeDtypeStruct((NC, ND), _f32),
    ),
    mesh=plsc.VectorSubcoreMesh(core_axis_name="c", subcore_axis_name="s"),
    compiler_params=pltpu.CompilerParams(use_tc_tiling_on_sc=False, needs_layout_passes=False),
    scratch_types=[
        pltpu.VMEM_SHARED((N, D), _f32),
        pltpu.VMEM_SHARED((ND,), _f32),
        pltpu.VMEM((N,), _f32),
        pltpu.VMEM((N,), _f32),
        pltpu.VMEM((2, W, 2, KB), jnp.int32),
        pltpu.VMEM((2 * KB, D), _f32),
        pltpu.VMEM((2, KB), _f32),
        pltpu.VMEM((DSTRIPE,), _f32),
        pltpu.SemaphoreType.DMA,
        pltpu.SemaphoreType.DMA,
        pltpu.SemaphoreType.DMA,
        pltpu.SemaphoreType.DMA,
    ],
)


def _fuse_body(accA, dnA, accB, dnB, biasA, biasB, W1, b1, W2, z_o, att_o):
    dA = (dnA[0, :N] + dnA[1, :N]).reshape(N, 1)
    dB = (dnB[0, :N] + dnB[1, :N]).reshape(N, 1)
    dA = jnp.where(dA == 0, _f32(1.0), dA)
    dB = jnp.where(dB == 0, _f32(1.0), dB)
    zA = (accA[0] + accA[1]) / dA + biasA[...]
    zB = (accB[0] + accB[1]) / dB + biasB[...]
    zA = jnp.where(zA > 0, zA, jnp.exp(zA) - _f32(1.0))
    zB = jnp.where(zB > 0, zB, jnp.exp(zB) - _f32(1.0))
    sA = jnp.dot(
        jnp.tanh(jnp.dot(zA, W1[...], preferred_element_type=_f32) + b1[...]),
        W2[...], preferred_element_type=_f32)
    sB = jnp.dot(
        jnp.tanh(jnp.dot(zB, W1[...], preferred_element_type=_f32) + b1[...]),
        W2[...], preferred_element_type=_f32)
    wA = jnp.mean(sA)
    wB = jnp.mean(sB)
    m = jnp.maximum(wA, wB)
    eA = jnp.exp(wA - m)
    eB = jnp.exp(wB - m)
    aA = eA / (eA + eB)
    aB = eB / (eA + eB)
    z_o[...] = aA * zA + aB * zB
    att_o[...] = jnp.concatenate(
        [jnp.broadcast_to(aA, (1, 1)), jnp.broadcast_to(aB, (1, 1))], axis=1)


_fuse = pl.pallas_call(
    _fuse_body,
    out_shape=(
        jax.ShapeDtypeStruct((N, D), _f32),
        jax.ShapeDtypeStruct((1, R), _f32),
    ),
)


def kernel(dst_feat, src_feat_A, src_feat_B, edge_index_A, edge_index_B,
           W_gat_A, attn_l_A, attn_r_A, bias_A,
           W_gat_B, attn_l_B, attn_r_B, bias_B,
           W1, b1, W2):
    srcA = edge_index_A[0].astype(jnp.int32).reshape(NW, NB, 1, KB)
    dstA = edge_index_A[1].astype(jnp.int32).reshape(NW, NB, 1, KB)
    srcB = edge_index_B[0].astype(jnp.int32).reshape(NW, NB, 1, KB)
    dstB = edge_index_B[1].astype(jnp.int32).reshape(NW, NB, 1, KB)
    sdA = jnp.concatenate([srcA, dstA], axis=2)
    sdB = jnp.concatenate([srcB, dstB], axis=2)

    hsA, hsB, sc4 = _proj(
        dst_feat, src_feat_A, src_feat_B,
        W_gat_A, attn_l_A.reshape(1, D), attn_r_A.reshape(1, D),
        W_gat_B, attn_l_B.reshape(1, D), attn_r_B.reshape(1, D))
    elA = sc4[:, 0]
    erA = sc4[:, 1]
    elB = sc4[:, 2]
    erB = sc4[:, 3]

    accA, denA, accB, denB = _edge(hsA, elA, erA, sdA,
                                   hsB, elB, erB, sdB)
    z, att = _fuse(accA, denA, accB, denB,
                   bias_A.reshape(1, D), bias_B.reshape(1, D),
                   W1, b1.reshape(1, SEM_H), W2)
    return z, att.reshape(R)


# speedup vs baseline: 1.0154x; 1.0020x over previous
import jax
import jax.numpy as jnp
from jax import lax
from jax.experimental import pallas as pl
from jax.experimental.pallas import tpu as pltpu
from jax.experimental.pallas import tpu_sc as plsc

N = 10000
E = 320000
D = 128
SEM_H = 128
R = 2

NC = 2
NS = 16
NW = NC * NS
L = 16

EPW = E // NW
KB = 80
NB = EPW // KB
VB = KB // L
W = 25
NBW = NB // W

STRIPE = N // NS
DSTRIPE = 640
ND = NS * DSTRIPE

_f32 = jnp.float32


def _proj_body(dstf, srcA, srcB, WA, alA, arA, WB, alB, arB,
               hsA_o, hsB_o, sc_o):
    hsA = jnp.dot(srcA[...], WA[...], preferred_element_type=_f32)
    hsB = jnp.dot(srcB[...], WB[...], preferred_element_type=_f32)
    hdA = jnp.dot(dstf[...], WA[...], preferred_element_type=_f32)
    hdB = jnp.dot(dstf[...], WB[...], preferred_element_type=_f32)
    hsA_o[...] = hsA
    hsB_o[...] = hsB
    elA = jnp.sum(hsA * alA[...], axis=1, keepdims=True)
    erA = jnp.sum(hdA * arA[...], axis=1, keepdims=True)
    elB = jnp.sum(hsB * alB[...], axis=1, keepdims=True)
    erB = jnp.sum(hdB * arB[...], axis=1, keepdims=True)
    sc_o[...] = jnp.concatenate([elA, erA, elB, erB], axis=1)


_proj = pl.pallas_call(
    _proj_body,
    out_shape=(
        jax.ShapeDtypeStruct((N, D), _f32),
        jax.ShapeDtypeStruct((N, D), _f32),
        jax.ShapeDtypeStruct((N, 4), _f32),
    ),
)


def _edge_body(hsA, elA, erA, sdA, hsB, elB, erB, sdB,
               accA_o, denA_o, accB_o, denB_o,
               acc_sh, den_sh, el_ts, er_ts, sdw2, rows, wbuf, zbuf,
               gsem, isem, asem, ssem):
    cid = lax.axis_index("c")
    sid = lax.axis_index("s")
    wid = cid * NS + sid
    zv = jnp.zeros((L,), _f32)

    for hs_hbm, el_hbm, er_hbm, sd_hbm, acc_o, den_o in (
        (hsA, elA, erA, sdA, accA_o, denA_o),
        (hsB, elB, erB, sdB, accB_o, denB_o),
    ):
        pltpu.sync_copy(el_hbm, el_ts)
        pltpu.sync_copy(er_hbm, er_ts)

        @plsc.parallel_loop(0, 2 * KB, step=1, unroll=4)
        def _zr(i):
            for cc in range(D // L):
                rows[i, pl.ds(cc * L, L)] = zv

        def _zz(i, c):
            zbuf[pl.ds(i * L, L)] = zv
            return c
        lax.fori_loop(0, DSTRIPE // L, _zz, 0)

        base = sid * STRIPE
        for t, ln in ((0, 160), (160, 160), (320, 160), (480, 145)):
            pltpu.sync_copy(rows.at[pl.ds(0, ln)],
                            acc_sh.at[pl.ds(base + t, ln)])
        pltpu.sync_copy(zbuf, den_sh.at[pl.ds(sid * DSTRIPE, DSTRIPE)])

        pltpu.sync_copy(sd_hbm.at[wid, pl.ds(0, W)], sdw2.at[0])
        pltpu.async_copy(sd_hbm.at[wid, pl.ds(W, W)], sdw2.at[1], isem)

        plsc.subcore_barrier()

        pltpu.async_copy(hs_hbm.at[sdw2.at[0, 0, 0]], rows.at[pl.ds(0, KB)],
                         gsem)

        def _batch(g, c):
            par = lax.rem(g, 2)
            pob = par * KB
            parn = lax.rem(g + 1, 2)
            k = lax.div(g, W)
            j = lax.rem(g, W)
            kpar = lax.rem(k, 2)
            kparn = 1 - kpar

            @pl.when(g + 1 < NB)
            def _():
                @pl.when(g >= 1)
                def _():
                    bp = jnp.where(j == 0, kparn, kpar)
                    jp = lax.rem(j - 1 + W, W)
                    pltpu.make_async_copy(rows.at[pl.ds(parn * KB, KB)],
                                          acc_sh.at[sdw2.at[bp, jp, 1]],
                                          ssem).wait()

                @pl.when(j == W - 1)
                def _():
                    pltpu.make_async_copy(
                        sd_hbm.at[wid, pl.ds((k + 1) * W, W)],
                        sdw2.at[kparn], isem).wait()

                bn = jnp.where(j == W - 1, kparn, kpar)
                jn = lax.rem(j + 1, W)
                pltpu.async_copy(hs_hbm.at[sdw2.at[bn, jn, 0]],
                                 rows.at[pl.ds(parn * KB, KB)], gsem)

            pltpu.make_async_copy(hs_hbm.at[sdw2.at[kpar, j, 0]],
                                  rows.at[pl.ds(pob, KB)], gsem).wait()

            @pl.when(g >= 2)
            def _():
                bp2 = jnp.where(j <= 1, kparn, kpar)
                jp2 = lax.rem(j - 2 + W, W)
                pltpu.make_async_copy(wbuf.at[par],
                                      den_sh.at[sdw2.at[bp2, jp2, 1]],
                                      asem).wait()

            @pl.when((j == 2) & (k + 1 < NBW) & (k >= 1))
            def _():
                pltpu.async_copy(sd_hbm.at[wid, pl.ds((k + 1) * W, W)],
                                 sdw2.at[kparn], isem)

            for v in range(VB):
                sidx = sdw2[kpar, j, 0, pl.ds(v * L, L)]
                didx = sdw2[kpar, j, 1, pl.ds(v * L, L)]
                e = plsc.load_gather(el_ts, [sidx]) + plsc.load_gather(er_ts, [didx])
                e = jnp.where(e > 0, e, _f32(0.2) * e)
                w = jnp.exp(e)
                wbuf[par, pl.ds(v * L, L)] = w

            pltpu.async_copy(wbuf.at[par], den_sh.at[sdw2.at[kpar, j, 1]],
                             asem, add=True)

            @plsc.parallel_loop(0, KB, step=1, unroll=4)
            def _scale(jj):
                pv = jnp.broadcast_to(par, (L,)).astype(jnp.int32)
                jv = jnp.broadcast_to(jj, (L,)).astype(jnp.int32)
                wsp = plsc.load_gather(wbuf, [pv, jv])
                for cc in range(D // L):
                    sl = pl.ds(cc * L, L)
                    rows[pob + jj, sl] = rows[pob + jj, sl] * wsp

            pltpu.async_copy(rows.at[pl.ds(pob, KB)],
                             acc_sh.at[sdw2.at[kpar, j, 1]], ssem, add=True)

            return c
        lax.fori_loop(0, NB, _batch, 0)

        pltpu.make_async_copy(wbuf.at[0], den_sh.at[sdw2.at[0, W - 1, 1]],
                              asem).wait()
        pltpu.make_async_copy(wbuf.at[1], den_sh.at[sdw2.at[0, W - 2, 1]],
                              asem).wait()
        pltpu.make_async_copy(rows.at[pl.ds(0, KB)],
                              acc_sh.at[sdw2.at[0, W - 1, 1]], ssem).wait()
        pltpu.make_async_copy(rows.at[pl.ds(KB, KB)],
                              acc_sh.at[sdw2.at[0, W - 2, 1]], ssem).wait()
        plsc.subcore_barrier()
        pltpu.sync_copy(acc_sh.at[pl.ds(base, STRIPE)],
                        acc_o.at[cid, pl.ds(base, STRIPE)])

        @pl.when(sid == 0)
        def _():
            pltpu.sync_copy(den_sh, den_o.at[cid])

        plsc.subcore_barrier()


_edge = pl.kernel(
    _edge_body,
    out_type=(
        jax.ShapeDtypeStruct((NC, N, D), _f32),
        jax.ShapeDtypeStruct((NC, ND), _f32),
        jax.ShapeDtypeStruct((NC, N, D), _f32),
        jax.ShapeDtypeStruct((NC, ND), _f32),
    ),
    mesh=plsc.VectorSubcoreMesh(core_axis_name="c", subcore_axis_name="s"),
    compiler_params=pltpu.CompilerParams(use_tc_tiling_on_sc=False, needs_layout_passes=False),
    scratch_types=[
        pltpu.VMEM_SHARED((N, D), _f32),
        pltpu.VMEM_SHARED((ND,), _f32),
        pltpu.VMEM((N,), _f32),
        pltpu.VMEM((N,), _f32),
        pltpu.VMEM((2, W, 2, KB), jnp.int32),
        pltpu.VMEM((2 * KB, D), _f32),
        pltpu.VMEM((2, KB), _f32),
        pltpu.VMEM((DSTRIPE,), _f32),
        pltpu.SemaphoreType.DMA,
        pltpu.SemaphoreType.DMA,
        pltpu.SemaphoreType.DMA,
        pltpu.SemaphoreType.DMA,
    ],
)


def _fuse_body(accA, dnA, accB, dnB, biasA, biasB, W1, b1, W2, z_o, att_o):
    dA = (dnA[0, :N] + dnA[1, :N]).reshape(N, 1)
    dB = (dnB[0, :N] + dnB[1, :N]).reshape(N, 1)
    dA = jnp.where(dA == 0, _f32(1.0), dA)
    dB = jnp.where(dB == 0, _f32(1.0), dB)
    zA = (accA[0] + accA[1]) / dA + biasA[...]
    zB = (accB[0] + accB[1]) / dB + biasB[...]
    zA = jnp.where(zA > 0, zA, jnp.exp(zA) - _f32(1.0))
    zB = jnp.where(zB > 0, zB, jnp.exp(zB) - _f32(1.0))
    sA = jnp.dot(
        jnp.tanh(jnp.dot(zA, W1[...], preferred_element_type=_f32) + b1[...]),
        W2[...], preferred_element_type=_f32)
    sB = jnp.dot(
        jnp.tanh(jnp.dot(zB, W1[...], preferred_element_type=_f32) + b1[...]),
        W2[...], preferred_element_type=_f32)
    wA = jnp.mean(sA)
    wB = jnp.mean(sB)
    m = jnp.maximum(wA, wB)
    eA = jnp.exp(wA - m)
    eB = jnp.exp(wB - m)
    aA = eA / (eA + eB)
    aB = eB / (eA + eB)
    z_o[...] = aA * zA + aB * zB
    att_o[...] = jnp.concatenate(
        [jnp.broadcast_to(aA, (1, 1)), jnp.broadcast_to(aB, (1, 1))], axis=1)


_fuse = pl.pallas_call(
    _fuse_body,
    out_shape=(
        jax.ShapeDtypeStruct((N, D), _f32),
        jax.ShapeDtypeStruct((1, R), _f32),
    ),
)


def kernel(dst_feat, src_feat_A, src_feat_B, edge_index_A, edge_index_B,
           W_gat_A, attn_l_A, attn_r_A, bias_A,
           W_gat_B, attn_l_B, attn_r_B, bias_B,
           W1, b1, W2):
    srcA = edge_index_A[0].astype(jnp.int32).reshape(NW, NB, 1, KB)
    dstA = edge_index_A[1].astype(jnp.int32).reshape(NW, NB, 1, KB)
    srcB = edge_index_B[0].astype(jnp.int32).reshape(NW, NB, 1, KB)
    dstB = edge_index_B[1].astype(jnp.int32).reshape(NW, NB, 1, KB)
    sdA = jnp.concatenate([srcA, dstA], axis=2)
    sdB = jnp.concatenate([srcB, dstB], axis=2)

    hsA, hsB, sc4 = _proj(
        dst_feat, src_feat_A, src_feat_B,
        W_gat_A, attn_l_A.reshape(1, D), attn_r_A.reshape(1, D),
        W_gat_B, attn_l_B.reshape(1, D), attn_r_B.reshape(1, D))
    elA = sc4[:, 0]
    erA = sc4[:, 1]
    elB = sc4[:, 2]
    erB = sc4[:, 3]

    accA, denA, accB, denB = _edge(hsA, elA, erA, sdA,
                                   hsB, elB, erB, sdB)
    z, att = _fuse(accA, denA, accB, denB,
                   bias_A.reshape(1, D), bias_B.reshape(1, D),
                   W1, b1.reshape(1, SEM_H), W2)
    return z, att.reshape(R)
